# baseline (speedup 1.0000x reference)
import jax
import jax.numpy as jnp
from jax import lax
from jax.experimental import pallas as pl
from jax.sharding import PartitionSpec as P

try:
    _shard_map = jax.shard_map
except AttributeError:
    from jax.experimental.shard_map import shard_map as _shard_map

_PROTO_NUM = 1024
_PROTO_DIM = 256
_BLOCK_ROWS = 1024


def _vq_block(x_ref, cb_ref, proto_ref, resid_ref, loss_ref):
    x = x_ref[...]
    cb = cb_ref[...]
    xn = jnp.sum(x * x, axis=1, keepdims=True)
    cn = jnp.sum(cb * cb, axis=1)
    cross = lax.dot_general(
        x, cb, (((1,), (1,)), ((), ())), preferred_element_type=jnp.float32
    )
    dist = xn + cn[None, :] - 2.0 * cross
    idx = jnp.argmin(dist, axis=1)
    oh = (
        lax.broadcasted_iota(jnp.int32, (x.shape[0], _PROTO_NUM), 1)
        == idx[:, None]
    ).astype(jnp.float32)
    proto = lax.dot_general(
        oh, cb, (((1,), (0,)), ((), ())), preferred_element_type=jnp.float32
    )
    resid = x - proto
    proto_ref[...] = proto
    resid_ref[...] = resid

    @pl.when(pl.program_id(0) == 0)
    def _init():
        loss_ref[...] = jnp.zeros_like(loss_ref)

    loss_ref[...] += jnp.sum(resid * resid).reshape(1, 1)


def _vq_shard(xf, codebook):
    n_rows = xf.shape[0]
    grid = n_rows // _BLOCK_ROWS

    proto, resid, loss_sum = pl.pallas_call(
        _vq_block,
        grid=(grid,),
        in_specs=[
            pl.BlockSpec((_BLOCK_ROWS, _PROTO_DIM), lambda i: (i, 0)),
            pl.BlockSpec((_PROTO_NUM, _PROTO_DIM), lambda i: (0, 0)),
        ],
        out_specs=[
            pl.BlockSpec((_BLOCK_ROWS, _PROTO_DIM), lambda i: (i, 0)),
            pl.BlockSpec((_BLOCK_ROWS, _PROTO_DIM), lambda i: (i, 0)),
            pl.BlockSpec((1, 1), lambda i: (0, 0)),
        ],
        out_shape=[
            jax.ShapeDtypeStruct((n_rows, _PROTO_DIM), jnp.float32),
            jax.ShapeDtypeStruct((n_rows, _PROTO_DIM), jnp.float32),
            jax.ShapeDtypeStruct((1, 1), jnp.float32),
        ],
    )(xf, codebook)
    loss_sum = jax.lax.psum(loss_sum.reshape(()), "d")
    return proto, resid, loss_sum


def kernel(x, codebook):
    x_shape = x.shape
    xf = x.reshape(-1, _PROTO_DIM)
    n_rows = xf.shape[0]

    n_shards = 2 if jax.device_count() >= 2 and n_rows % (2 * _BLOCK_ROWS) == 0 else 1
    mesh = jax.make_mesh((n_shards,), ("d",))
    xf = jax.reshard(xf, jax.sharding.NamedSharding(mesh, P("d", None)))
    codebook = jax.reshard(
        codebook, jax.sharding.NamedSharding(mesh, P(None, None))
    )
    proto, resid, loss_sum = _shard_map(
        _vq_shard,
        mesh=mesh,
        in_specs=(P("d", None), P(None, None)),
        out_specs=(P("d", None), P("d", None), P()),
        check_vma=False,
    )(xf, codebook)

    m = loss_sum / (n_rows * _PROTO_DIM)
    loss = m + 0.25 * m
    return (
        proto.reshape(x_shape),
        resid.reshape(x_shape),
        loss,
    )


# speedup vs baseline: 1.1199x; 1.1199x over previous
import jax
import jax.numpy as jnp
from jax import lax
from jax.experimental import pallas as pl
from jax.sharding import PartitionSpec as P

try:
    _shard_map = jax.shard_map
except AttributeError:
    from jax.experimental.shard_map import shard_map as _shard_map

_PROTO_NUM = 1024
_PROTO_DIM = 256
_BLOCK_ROWS = 1024


def _vq_block(x_ref, cb_ref, proto_ref, resid_ref, loss_ref):
    x = x_ref[...]
    cb = cb_ref[...]
    xn = jnp.sum(x * x, axis=1, keepdims=True)
    cn = jnp.sum(cb * cb, axis=1)
    cross = lax.dot_general(
        x, cb, (((1,), (1,)), ((), ())), preferred_element_type=jnp.float32
    )
    dist = xn + cn[None, :] - 2.0 * cross
    idx = jnp.argmin(dist, axis=1)
    oh = (
        lax.broadcasted_iota(jnp.int32, (x.shape[0], _PROTO_NUM), 1)
        == idx[:, None]
    ).astype(jnp.float32)
    proto = lax.dot_general(
        oh, cb, (((1,), (0,)), ((), ())), preferred_element_type=jnp.float32
    )
    resid = x - proto
    proto_ref[...] = proto
    resid_ref[...] = resid

    @pl.when(pl.program_id(0) == 0)
    def _init():
        loss_ref[...] = jnp.zeros_like(loss_ref)

    loss_ref[...] += jnp.sum(resid * resid).reshape(1, 1)


def _vq_shard(xf, codebook):
    n_rows = xf.shape[0]
    grid = n_rows // _BLOCK_ROWS

    proto, resid, loss_sum = pl.pallas_call(
        _vq_block,
        grid=(grid,),
        in_specs=[
            pl.BlockSpec((_BLOCK_ROWS, _PROTO_DIM), lambda i: (i, 0)),
            pl.BlockSpec((_PROTO_NUM, _PROTO_DIM), lambda i: (0, 0)),
        ],
        out_specs=[
            pl.BlockSpec((_BLOCK_ROWS, _PROTO_DIM), lambda i: (i, 0)),
            pl.BlockSpec((_BLOCK_ROWS, _PROTO_DIM), lambda i: (i, 0)),
            pl.BlockSpec((1, 1), lambda i: (0, 0)),
        ],
        out_shape=[
            jax.ShapeDtypeStruct((n_rows, _PROTO_DIM), jnp.float32),
            jax.ShapeDtypeStruct((n_rows, _PROTO_DIM), jnp.float32),
            jax.ShapeDtypeStruct((1, 1), jnp.float32),
        ],
    )(xf, codebook)
    return proto, resid, loss_sum


def kernel(x, codebook):
    x_shape = x.shape
    xf = x.reshape(-1, _PROTO_DIM)
    n_rows = xf.shape[0]

    n_shards = 2 if jax.device_count() >= 2 and n_rows % (2 * _BLOCK_ROWS) == 0 else 1
    mesh = jax.make_mesh((n_shards,), ("d",))
    xf = jax.reshard(xf, jax.sharding.NamedSharding(mesh, P("d", None)))
    codebook = jax.reshard(
        codebook, jax.sharding.NamedSharding(mesh, P(None, None))
    )
    proto, resid, loss_sum = _shard_map(
        _vq_shard,
        mesh=mesh,
        in_specs=(P("d", None), P(None, None)),
        out_specs=(P("d", None), P("d", None), P("d", None)),
        check_vma=False,
    )(xf, codebook)

    m = jnp.sum(loss_sum) / (n_rows * _PROTO_DIM)
    loss = m + 0.25 * m
    return (
        proto.reshape(x_shape),
        resid.reshape(x_shape),
        loss,
    )


# speedup vs baseline: 3.8868x; 3.4707x over previous
import jax
import jax.numpy as jnp
from jax import lax
from jax.experimental import pallas as pl
from jax.experimental.pallas import tpu as pltpu

_PROTO_NUM = 1024
_PROTO_DIM = 256
_BLOCK_ROWS = 2048


def _vq_block(x_ref, cb_ref, proto_ref, resid_ref, loss_ref, acc_ref):
    x = x_ref[...]
    cb = cb_ref[...]
    xn = jnp.sum(x * x, axis=1, keepdims=True)
    cn = jnp.sum(cb * cb, axis=1)
    cross = lax.dot_general(
        x, cb, (((1,), (1,)), ((), ())), preferred_element_type=jnp.float32
    )
    dist = xn + cn[None, :] - 2.0 * cross
    idx = jnp.argmin(dist, axis=1)
    oh = (
        lax.broadcasted_iota(jnp.int32, (x.shape[0], _PROTO_NUM), 1)
        == idx[:, None]
    ).astype(jnp.float32)
    proto = lax.dot_general(
        oh, cb, (((1,), (0,)), ((), ())), preferred_element_type=jnp.float32
    )
    resid = x - proto
    proto_ref[...] = proto
    resid_ref[...] = resid

    @pl.when(pl.program_id(0) == 0)
    def _init():
        acc_ref[...] = jnp.zeros_like(acc_ref)

    rr = resid * resid
    acc_ref[...] += jnp.sum(rr.reshape(-1, 8, 128), axis=0)

    @pl.when(pl.program_id(0) == pl.num_programs(0) - 1)
    def _finish():
        loss_ref[...] = jnp.sum(acc_ref[...]).reshape(1, 1)


def _vq_shard(xf, codebook):
    n_rows = xf.shape[0]
    grid = n_rows // _BLOCK_ROWS

    proto, resid, loss_sum = pl.pallas_call(
        _vq_block,
        grid=(grid,),
        in_specs=[
            pl.BlockSpec((_BLOCK_ROWS, _PROTO_DIM), lambda i: (i, 0)),
            pl.BlockSpec((_PROTO_NUM, _PROTO_DIM), lambda i: (0, 0)),
        ],
        out_specs=[
            pl.BlockSpec((_BLOCK_ROWS, _PROTO_DIM), lambda i: (i, 0)),
            pl.BlockSpec((_BLOCK_ROWS, _PROTO_DIM), lambda i: (i, 0)),
            pl.BlockSpec((1, 1), lambda i: (0, 0)),
        ],
        out_shape=[
            jax.ShapeDtypeStruct((n_rows, _PROTO_DIM), jnp.float32),
            jax.ShapeDtypeStruct((n_rows, _PROTO_DIM), jnp.float32),
            jax.ShapeDtypeStruct((1, 1), jnp.float32),
        ],
        scratch_shapes=[pltpu.VMEM((8, 128), jnp.float32)],
    )(xf, codebook)
    return proto, resid, loss_sum


def kernel(x, codebook):
    x_shape = x.shape
    xf = x.reshape(-1, _PROTO_DIM)
    n_rows = xf.shape[0]

    proto, resid, loss_sum = _vq_shard(xf, codebook)

    m = jnp.sum(loss_sum) / (n_rows * _PROTO_DIM)
    loss = m + 0.25 * m
    return (
        proto.reshape(x_shape),
        resid.reshape(x_shape),
        loss,
    )


# speedup vs baseline: 4.0671x; 1.0464x over previous
import jax
import jax.numpy as jnp
from jax import lax
from jax.experimental import pallas as pl
from jax.experimental.pallas import tpu as pltpu

_PROTO_NUM = 1024
_PROTO_DIM = 256
_BLOCK_ROWS = 3072


def _vq_block(x_ref, cb_ref, proto_ref, resid_ref, loss_ref, acc_ref):
    x = x_ref[...]
    cb = cb_ref[...]
    xn = jnp.sum(x * x, axis=1, keepdims=True)
    cn = jnp.sum(cb * cb, axis=1)
    cross = lax.dot_general(
        x, cb, (((1,), (1,)), ((), ())), preferred_element_type=jnp.float32
    )
    dist = xn + cn[None, :] - 2.0 * cross
    iota = lax.broadcasted_iota(jnp.int32, dist.shape, 1)
    minv = jnp.min(dist, axis=1, keepdims=True)
    cand = jnp.where(dist == minv, iota, _PROTO_NUM)
    idx = jnp.min(cand, axis=1, keepdims=True)
    oh = jnp.where(cand == idx, 1.0, 0.0)
    proto = lax.dot_general(
        oh, cb, (((1,), (0,)), ((), ())), preferred_element_type=jnp.float32
    )
    resid = x - proto
    proto_ref[...] = proto
    resid_ref[...] = resid

    @pl.when(pl.program_id(0) == 0)
    def _init():
        acc_ref[...] = jnp.zeros_like(acc_ref)

    acc_ref[...] += jnp.sum(minv.reshape(-1, 8, 128), axis=0)

    @pl.when(pl.program_id(0) == pl.num_programs(0) - 1)
    def _finish():
        loss_ref[...] = jnp.sum(acc_ref[...]).reshape(1, 1)


def _vq_shard(xf, codebook):
    n_rows = xf.shape[0]
    grid = n_rows // _BLOCK_ROWS

    proto, resid, loss_sum = pl.pallas_call(
        _vq_block,
        grid=(grid,),
        in_specs=[
            pl.BlockSpec((_BLOCK_ROWS, _PROTO_DIM), lambda i: (i, 0)),
            pl.BlockSpec((_PROTO_NUM, _PROTO_DIM), lambda i: (0, 0)),
        ],
        out_specs=[
            pl.BlockSpec((_BLOCK_ROWS, _PROTO_DIM), lambda i: (i, 0)),
            pl.BlockSpec((_BLOCK_ROWS, _PROTO_DIM), lambda i: (i, 0)),
            pl.BlockSpec((1, 1), lambda i: (0, 0)),
        ],
        out_shape=[
            jax.ShapeDtypeStruct((n_rows, _PROTO_DIM), jnp.float32),
            jax.ShapeDtypeStruct((n_rows, _PROTO_DIM), jnp.float32),
            jax.ShapeDtypeStruct((1, 1), jnp.float32),
        ],
        scratch_shapes=[pltpu.VMEM((8, 128), jnp.float32)],
    )(xf, codebook)
    return proto, resid, loss_sum


def kernel(x, codebook):
    x_shape = x.shape
    xf = x.reshape(-1, _PROTO_DIM)
    n_rows = xf.shape[0]

    proto, resid, loss_sum = _vq_shard(xf, codebook)

    m = jnp.sum(loss_sum) / (n_rows * _PROTO_DIM)
    loss = m + 0.25 * m
    return (
        proto.reshape(x_shape),
        resid.reshape(x_shape),
        loss,
    )
